# baseline (speedup 1.0000x reference)
import functools

import jax
import jax.numpy as jnp
from jax import lax
from jax.experimental import pallas as pl
from jax.experimental.pallas import tpu as pltpu
from jax.experimental.pallas import tpu_sc as plsc

TQ = 2048
TK = 8192
D = 768
KNN = 4
EPS = 1e-6

KB = 1024
NKB = TK // KB
NCAND = NKB * KNN

NC = 2
NS = 16
NW = NC * NS
QPW = TQ // NW
SUB = 16
NSUB = QPW // SUB
DCH = D // 16


def _simtop_body(nq, q_ref, k_ref, idx_ref, qn_ref, vals_ref, cidx_ref):
    kb = pl.program_id(0)

    @pl.when(kb == 0)
    def _normq():
        q = q_ref[...]
        qn_ref[...] = q / (jnp.sqrt(jnp.sum(q * q, axis=1, keepdims=True))
                           + EPS)

    kblk = k_ref[...]
    kn = kblk / (jnp.sqrt(jnp.sum(kblk * kblk, axis=1, keepdims=True)) + EPS)
    sim = lax.dot_general(
        kn, qn_ref[...],
        dimension_numbers=(((1,), (1,)), ((), ())),
        preferred_element_type=jnp.float32,
    )
    ri = lax.broadcasted_iota(jnp.int32, (3, KB), 1)
    wsel = lax.broadcasted_iota(jnp.int32, (3, KB), 0)
    w = jnp.where(wsel == 0, (ri % 256).astype(jnp.float32),
                  jnp.where(wsel == 1, (ri // 256).astype(jnp.float32), 1.0))
    neg = jnp.float32(-jnp.inf)
    m_prev = None
    tie = jnp.float32(0.0)
    for t in range(KNN):
        slot = NKB * t + kb
        if t == 0:
            m = jnp.max(sim, axis=0, keepdims=True)
        else:
            m = jnp.max(jnp.where(sim < m_prev, sim, neg), axis=0,
                        keepdims=True)
        eqf = jnp.where(sim == m, 1.0, 0.0)
        r = lax.dot_general(w, eqf,
                            dimension_numbers=(((1,), (0,)), ((), ())),
                            preferred_element_type=jnp.float32)
        vals_ref[slot] = m
        cidx_ref[slot] = (r[0:1, :].astype(jnp.int32)
                          + r[1:2, :].astype(jnp.int32) * 256 + kb * KB)
        tie = jnp.maximum(tie, jnp.max(r[2:3, :]))
        m_prev = m

    @pl.when(tie > 1.5)
    def _exact_tie_redo():
        row = lax.broadcasted_iota(jnp.int32, (KB, nq), 0)
        simx = sim
        for t in range(KNN):
            mt = jnp.max(simx, axis=0, keepdims=True)
            jt = jnp.min(jnp.where(simx == mt, row, KB), axis=0,
                         keepdims=True)
            vals_ref[NKB * t + kb] = mt
            cidx_ref[NKB * t + kb] = jt + kb * KB
            simx = jnp.where(row == jt, neg, simx)

    @pl.when(kb == NKB - 1)
    def _merge():
        vals = vals_ref[...].reshape(NCAND, nq)
        cidx = cidx_ref[...].reshape(NCAND, nq)
        r = lax.broadcasted_iota(jnp.int32, (NCAND, nq), 0)
        pos = (r % NKB) * KNN + (r // NKB)
        neg2 = jnp.float32(-jnp.inf)
        for t in range(KNN):
            m = jnp.max(vals, axis=0, keepdims=True)
            p = jnp.min(jnp.where(vals == m, pos, NCAND), axis=0,
                        keepdims=True)
            sel = pos == p
            idx_ref[t] = jnp.sum(jnp.where(sel, cidx, 0), axis=0,
                                 keepdims=True)
            vals = jnp.where(sel, neg2, vals)


def _simtop4(qn, kn):
    nq = qn.shape[0]
    return pl.pallas_call(
        functools.partial(_simtop_body, nq),
        grid=(NKB,),
        in_specs=[
            pl.BlockSpec((nq, D), lambda kb: (0, 0)),
            pl.BlockSpec((KB, D), lambda kb: (kb, 0)),
        ],
        out_specs=pl.BlockSpec((KNN, 1, nq), lambda kb: (0, 0, 0)),
        out_shape=jax.ShapeDtypeStruct((KNN, 1, nq), jnp.int32),
        scratch_shapes=[
            pltpu.VMEM((nq, D), jnp.float32),
            pltpu.VMEM((NCAND, 1, nq), jnp.float32),
            pltpu.VMEM((NCAND, 1, nq), jnp.int32),
        ],
        compiler_params=pltpu.CompilerParams(
            dimension_semantics=("arbitrary",),
        ),
    )(qn, kn)


def _gather_blend_body(nq, q_hbm, k_hbm, idx_hbm, a_hbm, out_hbm,
                       rows_a, rows_b, q_a, q_b, idxv, av,
                       sg_a, sg_b, sq_a, sq_b, so_a, so_b):
    qpw = nq // NW
    nsub = qpw // SUB
    wid = lax.axis_index("s") * NC + lax.axis_index("c")
    base = wid * qpw
    pltpu.sync_copy(a_hbm, av)
    hidx = [pltpu.async_copy(
        idx_hbm.at[pl.ds(t * nq + base, qpw)],
        idxv.at[pl.ds(t * qpw, qpw)], sg_a) for t in range(KNN)]
    for h in hidx:
        h.wait()
    a = av[...]
    s = (1.0 - a) * jnp.float32(1.0 / KNN)

    rows = (rows_a, rows_b)
    qbuf = (q_a, q_b)
    sg = (sg_a, sg_b)
    sq = (sq_a, sq_b)
    so = (so_a, so_b)

    def start(c):
        p = c % 2
        hg = [pltpu.async_copy(
            k_hbm.at[idxv.at[pl.ds(t * qpw + SUB * c, SUB)]],
            rows[p].at[pl.ds(t * SUB, SUB)], sg[p]) for t in range(KNN)]
        hq = pltpu.async_copy(
            q_hbm.at[pl.ds(base + SUB * c, SUB)], qbuf[p], sq[p])
        return hg, hq

    in_flight = {0: start(0)}
    out_flight = {}
    for c in range(nsub):
        p = c % 2
        hgs, hq = in_flight[c]
        for h in hgs:
            h.wait()
        hq.wait()
        if c + 1 < nsub:
            if c >= 1:
                out_flight[c - 1].wait()
            in_flight[c + 1] = start(c + 1)

        def qbody(q, carry):
            for d in range(DCH):
                sl = pl.ds(d * 16, 16)
                acc = (rows[p][q, sl] + rows[p][SUB + q, sl]
                       + rows[p][2 * SUB + q, sl] + rows[p][3 * SUB + q, sl])
                qbuf[p][q, sl] = acc * s + qbuf[p][q, sl] * a
            return carry

        lax.fori_loop(0, SUB, qbody, 0)
        out_flight[c] = pltpu.async_copy(
            qbuf[p], out_hbm.at[pl.ds(base + SUB * c, SUB)], so[p])
    out_flight[nsub - 2].wait()
    out_flight[nsub - 1].wait()


def _gather_blend(queries, keys, idx_flat, alpha16):
    nq = queries.shape[0]
    kern = pl.kernel(
        functools.partial(_gather_blend_body, nq),
        mesh=plsc.VectorSubcoreMesh(core_axis_name="c", subcore_axis_name="s"),
        out_type=jax.ShapeDtypeStruct((nq, D), jnp.float32),
        scratch_types=[
            pltpu.VMEM((KNN * SUB, D), jnp.float32),
            pltpu.VMEM((KNN * SUB, D), jnp.float32),
            pltpu.VMEM((SUB, D), jnp.float32),
            pltpu.VMEM((SUB, D), jnp.float32),
            pltpu.VMEM((KNN * (nq // NW),), jnp.int32),
            pltpu.VMEM((16,), jnp.float32),
            pltpu.SemaphoreType.DMA,
            pltpu.SemaphoreType.DMA,
            pltpu.SemaphoreType.DMA,
            pltpu.SemaphoreType.DMA,
            pltpu.SemaphoreType.DMA,
            pltpu.SemaphoreType.DMA,
        ],
    )
    return kern(queries, keys, idx_flat, alpha16)


def kernel(queries, keys, k, alpha):
    del k
    alpha16 = jnp.broadcast_to(alpha.astype(jnp.float32), (16,))
    hq = TQ // 2
    outs = []
    for h in range(2):
        qh = lax.slice_in_dim(queries, h * hq, (h + 1) * hq)
        idx3 = _simtop4(qh, keys)
        outs.append(_gather_blend(qh, keys, idx3.reshape(KNN * hq), alpha16))
    return jnp.concatenate(outs, axis=0)


# speedup vs baseline: 1.0440x; 1.0440x over previous
import functools

import jax
import jax.numpy as jnp
from jax import lax
from jax.experimental import pallas as pl
from jax.experimental.pallas import tpu as pltpu
from jax.experimental.pallas import tpu_sc as plsc

TQ = 2048
TK = 8192
D = 768
KNN = 4
EPS = 1e-6

KB = 1024
NKB = TK // KB
NCAND = NKB * KNN

NC = 2
NS = 16
NW = NC * NS
QPW = TQ // NW
SUB = 16
NSUB = QPW // SUB
DCH = D // 16


def _simtop_body(q_ref, k_ref, idx_ref, qn_ref, vals_ref, cidx_ref):
    kb = pl.program_id(0)

    @pl.when(kb == 0)
    def _normq():
        q = q_ref[...]
        qn_ref[...] = q / (jnp.sqrt(jnp.sum(q * q, axis=1, keepdims=True))
                           + EPS)

    kblk = k_ref[...]
    kn = kblk / (jnp.sqrt(jnp.sum(kblk * kblk, axis=1, keepdims=True)) + EPS)
    sim = lax.dot_general(
        kn, qn_ref[...],
        dimension_numbers=(((1,), (1,)), ((), ())),
        preferred_element_type=jnp.float32,
    )
    ri = lax.broadcasted_iota(jnp.int32, (3, KB), 1)
    wsel = lax.broadcasted_iota(jnp.int32, (3, KB), 0)
    w = jnp.where(wsel == 0, (ri % 256).astype(jnp.float32),
                  jnp.where(wsel == 1, (ri // 256).astype(jnp.float32), 1.0))
    neg = jnp.float32(-jnp.inf)
    m_prev = None
    tie = jnp.float32(0.0)
    for t in range(KNN):
        slot = NKB * t + kb
        if t == 0:
            m = jnp.max(sim, axis=0, keepdims=True)
        else:
            m = jnp.max(jnp.where(sim < m_prev, sim, neg), axis=0,
                        keepdims=True)
        eqf = jnp.where(sim == m, 1.0, 0.0)
        r = lax.dot_general(w, eqf,
                            dimension_numbers=(((1,), (0,)), ((), ())),
                            preferred_element_type=jnp.float32)
        vals_ref[slot] = m
        cidx_ref[slot] = (r[0:1, :].astype(jnp.int32)
                          + r[1:2, :].astype(jnp.int32) * 256 + kb * KB)
        tie = jnp.maximum(tie, jnp.max(r[2:3, :]))
        m_prev = m

    @pl.when(tie > 1.5)
    def _exact_tie_redo():
        row = lax.broadcasted_iota(jnp.int32, (KB, TQ), 0)
        simx = sim
        for t in range(KNN):
            mt = jnp.max(simx, axis=0, keepdims=True)
            jt = jnp.min(jnp.where(simx == mt, row, KB), axis=0,
                         keepdims=True)
            vals_ref[NKB * t + kb] = mt
            cidx_ref[NKB * t + kb] = jt + kb * KB
            simx = jnp.where(row == jt, neg, simx)

    @pl.when(kb == NKB - 1)
    def _merge():
        vals = vals_ref[...].reshape(NCAND, TQ)
        cidx = cidx_ref[...].reshape(NCAND, TQ)
        r = lax.broadcasted_iota(jnp.int32, (NCAND, TQ), 0)
        pos = (r % NKB) * KNN + (r // NKB)
        neg2 = jnp.float32(-jnp.inf)
        for t in range(KNN):
            m = jnp.max(vals, axis=0, keepdims=True)
            p = jnp.min(jnp.where(vals == m, pos, NCAND), axis=0,
                        keepdims=True)
            sel = pos == p
            idx_ref[t] = jnp.sum(jnp.where(sel, cidx, 0), axis=0,
                                 keepdims=True)
            vals = jnp.where(sel, neg2, vals)


def _simtop4(qn, kn):
    return pl.pallas_call(
        _simtop_body,
        grid=(NKB,),
        in_specs=[
            pl.BlockSpec((TQ, D), lambda kb: (0, 0)),
            pl.BlockSpec((KB, D), lambda kb: (kb, 0)),
        ],
        out_specs=pl.BlockSpec((KNN, 1, TQ), lambda kb: (0, 0, 0)),
        out_shape=jax.ShapeDtypeStruct((KNN, 1, TQ), jnp.int32),
        scratch_shapes=[
            pltpu.VMEM((TQ, D), jnp.float32),
            pltpu.VMEM((NCAND, 1, TQ), jnp.float32),
            pltpu.VMEM((NCAND, 1, TQ), jnp.int32),
        ],
        compiler_params=pltpu.CompilerParams(
            dimension_semantics=("arbitrary",),
        ),
    )(qn, kn)


def _gather_blend_body(q_hbm, k_hbm, idx_hbm, a_hbm, out_hbm,
                       rows_a, rows_b, q_a, q_b, idxv, av,
                       sg_a, sg_b, sq_a, sq_b, so_a, so_b):
    wid = lax.axis_index("s") * NC + lax.axis_index("c")
    base = wid * QPW
    pltpu.sync_copy(a_hbm, av)
    hidx = [pltpu.async_copy(
        idx_hbm.at[pl.ds(t * TQ + base, QPW)],
        idxv.at[pl.ds(t * QPW, QPW)], sg_a) for t in range(KNN)]
    for h in hidx:
        h.wait()
    a = av[...]
    s = (1.0 - a) * jnp.float32(1.0 / KNN)

    rows = (rows_a, rows_b)
    qbuf = (q_a, q_b)
    sg = (sg_a, sg_b)
    sq = (sq_a, sq_b)
    so = (so_a, so_b)

    def start(c):
        p = c % 2
        hg = [pltpu.async_copy(
            k_hbm.at[idxv.at[pl.ds(t * QPW + SUB * c, SUB)]],
            rows[p].at[pl.ds(t * SUB, SUB)], sg[p]) for t in range(KNN)]
        hq = pltpu.async_copy(
            q_hbm.at[pl.ds(base + SUB * c, SUB)], qbuf[p], sq[p])
        return hg, hq

    in_flight = {0: start(0)}
    out_flight = {}
    for c in range(NSUB):
        p = c % 2
        hgs, hq = in_flight[c]
        for h in hgs:
            h.wait()
        hq.wait()
        if c + 1 < NSUB:
            if c >= 1:
                out_flight[c - 1].wait()
            in_flight[c + 1] = start(c + 1)

        def qbody(q, carry):
            for d in range(DCH):
                sl = pl.ds(d * 16, 16)
                acc = (rows[p][q, sl] + rows[p][SUB + q, sl]
                       + rows[p][2 * SUB + q, sl] + rows[p][3 * SUB + q, sl])
                qbuf[p][q, sl] = acc * s + qbuf[p][q, sl] * a
            return carry

        lax.fori_loop(0, SUB, qbody, 0)
        out_flight[c] = pltpu.async_copy(
            qbuf[p], out_hbm.at[pl.ds(base + SUB * c, SUB)], so[p])
    out_flight[NSUB - 2].wait()
    out_flight[NSUB - 1].wait()


def _gather_blend(queries, keys, idx_flat, alpha16):
    kern = pl.kernel(
        _gather_blend_body,
        mesh=plsc.VectorSubcoreMesh(core_axis_name="c", subcore_axis_name="s"),
        out_type=jax.ShapeDtypeStruct((TQ, D), jnp.float32),
        scratch_types=[
            pltpu.VMEM((KNN * SUB, D), jnp.float32),
            pltpu.VMEM((KNN * SUB, D), jnp.float32),
            pltpu.VMEM((SUB, D), jnp.float32),
            pltpu.VMEM((SUB, D), jnp.float32),
            pltpu.VMEM((KNN * QPW,), jnp.int32),
            pltpu.VMEM((16,), jnp.float32),
            pltpu.SemaphoreType.DMA,
            pltpu.SemaphoreType.DMA,
            pltpu.SemaphoreType.DMA,
            pltpu.SemaphoreType.DMA,
            pltpu.SemaphoreType.DMA,
            pltpu.SemaphoreType.DMA,
        ],
    )
    return kern(queries, keys, idx_flat, alpha16)


def kernel(queries, keys, k, alpha):
    del k
    idx3 = _simtop4(queries, keys)
    idx_flat = idx3.reshape(KNN * TQ)
    alpha16 = jnp.broadcast_to(alpha.astype(jnp.float32), (16,))
    out = _gather_blend(queries, keys, idx_flat, alpha16)
    return out
